# baseline (speedup 1.0000x reference)
import functools

import jax
import jax.numpy as jnp
from jax import lax
from jax.experimental import pallas as pl
from jax.experimental.pallas import tpu as pltpu
from jax.experimental.pallas import tpu_sc as plsc

HID = 1024
FFN = 3584
E = 8
T = 128
FB = 896
NFB = FFN // FB

L = 16
NW = 32
TPW = T // NW
NCH = HID // L


def _all_reduce(v, op):
    lane = lax.iota(jnp.int32, L)
    for k in (1, 2, 4, 8):
        v = op(v, v.at[lane ^ k].get(mode="promise_in_bounds"))
    return v


def _router_sc(x_hbm, gw_hbm, comb_hbm, xv, gwv, combv):
    wid = lax.axis_index("s") * 2 + lax.axis_index("c")
    base = wid * TPW
    pltpu.sync_copy(x_hbm.at[pl.ds(base, TPW)], xv)
    pltpu.sync_copy(gw_hbm, gwv)
    lane = lax.iota(jnp.int32, L)
    for t in range(TPW):
        def body(i, accs):
            xc = xv[t, pl.ds(i * L, L)]
            return tuple(
                accs[e] + xc * gwv[e, pl.ds(i * L, L)] for e in range(E))
        accs = lax.fori_loop(
            0, NCH, body, tuple(jnp.zeros((L,), jnp.float32)
                                for _ in range(E)))
        logits = jnp.full((L,), -1e30, jnp.float32)
        for e in range(E):
            logits = jnp.where(lane == e, _all_reduce(accs[e], jnp.add),
                               logits)
        m1 = _all_reduce(logits, jnp.maximum)
        i1 = _all_reduce(jnp.where(logits == m1, lane, L), jnp.minimum)
        masked = jnp.where(lane == i1, -jnp.inf, logits)
        m2 = _all_reduce(masked, jnp.maximum)
        i2 = _all_reduce(jnp.where(masked == m2, lane, L), jnp.minimum)
        p1 = 1.0 / (1.0 + jnp.exp(m2 - m1))
        row = jnp.where(lane == i1, p1,
                        jnp.where(lane == i2, 1.0 - p1,
                                  jnp.zeros((L,), jnp.float32)))
        combv[t, :] = row
    pltpu.sync_copy(combv, comb_hbm.at[pl.ds(base, TPW)])


@functools.partial(
    pl.kernel,
    mesh=plsc.VectorSubcoreMesh(core_axis_name="c", subcore_axis_name="s"),
    out_type=jax.ShapeDtypeStruct((T, L), jnp.float32),
    scratch_types=[
        pltpu.VMEM((TPW, HID), jnp.float32),
        pltpu.VMEM((E, HID), jnp.float32),
        pltpu.VMEM((TPW, L), jnp.float32),
    ],
)
def _router(x_hbm, gw_hbm, comb_hbm, xv, gwv, combv):
    _router_sc(x_hbm, gw_hbm, comb_hbm, xv, gwv, combv)


def _moe_body(x_ref, comb_ref, w1_ref, w3_ref, w2_ref, out_ref):
    e = pl.program_id(0)
    f = pl.program_id(1)
    x = x_ref[...]

    lane = lax.broadcasted_iota(jnp.int32, (T, L), 1)
    combine = jnp.sum(jnp.where(lane == e, comb_ref[...], 0.0),
                      axis=1, keepdims=True)

    w1b = w1_ref[0]
    w3b = w3_ref[0]
    w2b = w2_ref[0]
    dn = (((1,), (1,)), ((), ()))
    h = jax.lax.dot_general(x, w1b, dn)
    g = jax.lax.dot_general(x, w3b, dn)
    act = (h * jax.nn.sigmoid(h)) * g
    act = act * combine
    outp = jax.lax.dot_general(act, w2b, dn)

    @pl.when(jnp.logical_and(e == 0, f == 0))
    def _init():
        out_ref[...] = jnp.zeros_like(out_ref)

    out_ref[...] += outp


def kernel(hidden_states, gate_w, w1, w3, w2):
    comb = _router(hidden_states, gate_w)
    return pl.pallas_call(
        _moe_body,
        grid=(E, NFB),
        in_specs=[
            pl.BlockSpec((T, HID), lambda e, f: (0, 0)),
            pl.BlockSpec((T, L), lambda e, f: (0, 0)),
            pl.BlockSpec((1, FB, HID), lambda e, f: (e, f, 0)),
            pl.BlockSpec((1, FB, HID), lambda e, f: (e, f, 0)),
            pl.BlockSpec((1, HID, FB), lambda e, f: (e, 0, f)),
        ],
        out_specs=pl.BlockSpec((T, HID), lambda e, f: (0, 0)),
        out_shape=jax.ShapeDtypeStruct((T, HID), hidden_states.dtype),
        compiler_params=pltpu.CompilerParams(
            dimension_semantics=("arbitrary", "arbitrary"),
        ),
    )(hidden_states, comb, w1, w3, w2)


# speedup vs baseline: 1.2378x; 1.2378x over previous
import jax
import jax.numpy as jnp
from jax import lax
from jax.experimental import pallas as pl
from jax.experimental.pallas import tpu as pltpu

HID = 1024
FFN = 3584
E = 8
T = 128
FB = 896
NFB = FFN // FB
CW = 128


def _moe_body(x_ref, gw_ref, w1_ref, w3_ref, w2_ref, out_ref, comb_ref):
    e = pl.program_id(0)
    f = pl.program_id(1)
    x = x_ref[...]
    dn = (((1,), (1,)), ((), ()))

    @pl.when(jnp.logical_and(e == 0, f == 0))
    def _init():
        logits = lax.dot_general(x, gw_ref[...], dn)
        iota = lax.broadcasted_iota(jnp.int32, (T, E), 1)
        v1 = jnp.max(logits, axis=1, keepdims=True)
        i1 = jnp.min(jnp.where(logits == v1, iota, E), axis=1, keepdims=True)
        masked = jnp.where(iota == i1, -jnp.inf, logits)
        v2 = jnp.max(masked, axis=1, keepdims=True)
        i2 = jnp.min(jnp.where(masked == v2, iota, E), axis=1, keepdims=True)
        p1 = jax.nn.sigmoid(v1 - v2)
        lanes = lax.broadcasted_iota(jnp.int32, (T, CW), 1)
        comb_ref[...] = jnp.where(lanes == i1, p1,
                                  jnp.where(lanes == i2, 1.0 - p1, 0.0))
        out_ref[...] = jnp.zeros_like(out_ref)

    onehot = (lax.broadcasted_iota(jnp.int32, (CW, 1), 0) == e
              ).astype(jnp.float32)
    combine = jnp.dot(comb_ref[...], onehot)

    w1b = w1_ref[0]
    w3b = w3_ref[0]
    w2b = w2_ref[0]
    h = lax.dot_general(x, w1b, dn)
    g = lax.dot_general(x, w3b, dn)
    act = (h * jax.nn.sigmoid(h)) * g
    act = act * combine
    out_ref[...] += lax.dot_general(act, w2b, dn)


def kernel(hidden_states, gate_w, w1, w3, w2):
    return pl.pallas_call(
        _moe_body,
        grid=(E, NFB),
        in_specs=[
            pl.BlockSpec((T, HID), lambda e, f: (0, 0)),
            pl.BlockSpec((E, HID), lambda e, f: (0, 0)),
            pl.BlockSpec((1, FB, HID), lambda e, f: (e, f, 0)),
            pl.BlockSpec((1, FB, HID), lambda e, f: (e, f, 0)),
            pl.BlockSpec((1, HID, FB), lambda e, f: (e, 0, f)),
        ],
        out_specs=pl.BlockSpec((T, HID), lambda e, f: (0, 0)),
        out_shape=jax.ShapeDtypeStruct((T, HID), hidden_states.dtype),
        scratch_shapes=[pltpu.VMEM((T, CW), jnp.float32)],
        compiler_params=pltpu.CompilerParams(
            dimension_semantics=("arbitrary", "arbitrary"),
        ),
    )(hidden_states, gate_w, w1, w3, w2)


# speedup vs baseline: 1.2538x; 1.0129x over previous
import jax
import jax.numpy as jnp
from jax import lax
from jax.experimental import pallas as pl
from jax.experimental.pallas import tpu as pltpu

HID = 1024
FFN = 3584
E = 8
T = 128
FB = 896
NFB = FFN // FB


def _moe_body(x_ref, gw_ref, w1_ref, w3_ref, w2_ref, out_ref):
    e = pl.program_id(0)
    f = pl.program_id(1)
    x = x_ref[...]
    dn = (((1,), (1,)), ((), ()))

    logits = lax.dot_general(x, gw_ref[...], dn)
    iota = lax.broadcasted_iota(jnp.int32, (T, E), 1)
    v1 = jnp.max(logits, axis=1, keepdims=True)
    i1 = jnp.min(jnp.where(logits == v1, iota, E), axis=1, keepdims=True)
    masked = jnp.where(iota == i1, -jnp.inf, logits)
    v2 = jnp.max(masked, axis=1, keepdims=True)
    i2 = jnp.min(jnp.where(masked == v2, iota, E), axis=1, keepdims=True)
    p1 = jax.nn.sigmoid(v1 - v2)
    combine = jnp.where(i1 == e, p1, jnp.where(i2 == e, 1.0 - p1, 0.0))

    w1b = w1_ref[0]
    w3b = w3_ref[0]
    w2b = w2_ref[0]
    h = lax.dot_general(x, w1b, dn)
    g = lax.dot_general(x, w3b, dn)
    act = (h * jax.nn.sigmoid(h)) * g
    act = act * combine
    outp = lax.dot_general(act, w2b, dn)

    @pl.when(jnp.logical_and(e == 0, f == 0))
    def _init():
        out_ref[...] = jnp.zeros_like(out_ref)

    out_ref[...] += outp


def kernel(hidden_states, gate_w, w1, w3, w2):
    return pl.pallas_call(
        _moe_body,
        grid=(E, NFB),
        in_specs=[
            pl.BlockSpec((T, HID), lambda e, f: (0, 0)),
            pl.BlockSpec((E, HID), lambda e, f: (0, 0)),
            pl.BlockSpec((1, FB, HID), lambda e, f: (e, f, 0)),
            pl.BlockSpec((1, FB, HID), lambda e, f: (e, f, 0)),
            pl.BlockSpec((1, HID, FB), lambda e, f: (e, 0, f)),
        ],
        out_specs=pl.BlockSpec((T, HID), lambda e, f: (0, 0)),
        out_shape=jax.ShapeDtypeStruct((T, HID), hidden_states.dtype),
        compiler_params=pltpu.CompilerParams(
            dimension_semantics=("arbitrary", "arbitrary"),
        ),
    )(hidden_states, gate_w, w1, w3, w2)
